# Initial kernel scaffold; baseline (speedup 1.0000x reference)
import jax
import jax.numpy as jnp
from jax.experimental import pallas as pl


def kernel(latents, prototypes):
    raise NotImplementedError("write your pallas kernel here")


# speedup vs baseline: 2.3800x; 2.3800x over previous
import functools

import jax
import jax.numpy as jnp
from jax import lax
from jax.experimental import pallas as pl
from jax.experimental.pallas import tpu as pltpu
from jax.experimental.pallas import tpu_sc as plsc

NUM_K = 8192
DIM = 256
ALPHA = 0.25
ENT_W = 0.01
BN = 256


def _vq_tc_body(lat_ref, ln_ref, pn_ref, proto_ref, idx_ref, loss_ref,
                colacc, sums, *, nsteps, n_total):
    i = pl.program_id(0)

    @pl.when(i == 0)
    def _init():
        colacc[...] = jnp.zeros_like(colacc)
        sums[0, 0] = 0.0

    lt = lat_ref[...]
    pt = proto_ref[...]
    mm = lax.dot_general(lt, pt, (((1,), (1,)), ((), ())),
                         preferred_element_type=jnp.float32)
    d = (ln_ref[...] + pn_ref[...]) - 2.0 * mm
    minv = jnp.min(d, axis=1, keepdims=True)
    jidx = lax.broadcasted_iota(jnp.int32, (BN, NUM_K), 1)
    idx = jnp.min(jnp.where(d == minv, jidx, NUM_K), axis=1, keepdims=True)
    idx_ref[...] = idx

    e = jnp.exp(minv - d)
    z = jnp.sum(e, axis=1, keepdims=True)
    colacc[...] += jnp.sum(e * (1.0 / z), axis=0, keepdims=True)
    sums[0, 0] += jnp.sum(minv)

    @pl.when(i == nsteps - 1)
    def _fin():
        s = colacc[...] * (1.0 / n_total) + 1e-6
        s = s * (1.0 / jnp.sum(s))
        ent = -jnp.sum(s * jnp.log(s))
        val = (sums[0, 0] * ((1.0 + ALPHA) / (n_total * DIM)) + ENT_W * ent)
        loss_ref[...] = jnp.reshape(val, (1, 1))


def _tc_call(latents, ln, pn, prototypes, interpret=False):
    n = latents.shape[0]
    nsteps = n // BN
    return pl.pallas_call(
        functools.partial(_vq_tc_body, nsteps=nsteps, n_total=n),
        grid=(nsteps,),
        in_specs=[
            pl.BlockSpec((BN, DIM), lambda i: (i, 0)),
            pl.BlockSpec((BN, 1), lambda i: (i, 0)),
            pl.BlockSpec((1, NUM_K), lambda i: (0, 0)),
            pl.BlockSpec((NUM_K, DIM), lambda i: (0, 0)),
        ],
        out_specs=[
            pl.BlockSpec((BN, 1), lambda i: (i, 0)),
            pl.BlockSpec((1, 1), lambda i: (0, 0)),
        ],
        out_shape=[
            jax.ShapeDtypeStruct((n, 1), jnp.int32),
            jax.ShapeDtypeStruct((1, 1), jnp.float32),
        ],
        scratch_shapes=[
            pltpu.VMEM((1, NUM_K), jnp.float32),
            pltpu.SMEM((1, 1), jnp.float32),
        ],
        interpret=interpret,
    )(latents, ln, pn, prototypes)


def _sc_gather(table, idx):
    n = idx.shape[0]
    info = plsc.get_sparse_core_info()
    nw = info.num_cores * info.num_subcores
    bpw = n // nw
    ch = 128
    mesh = plsc.VectorSubcoreMesh(core_axis_name="c", subcore_axis_name="s")

    @functools.partial(
        pl.kernel, mesh=mesh,
        out_type=jax.ShapeDtypeStruct((n, DIM), jnp.float32),
        scratch_types=[
            pltpu.VMEM((ch,), jnp.int32),
            pltpu.VMEM((ch, DIM), jnp.float32),
            pltpu.SemaphoreType.DMA,
        ],
    )
    def k(table_hbm, idx_hbm, out_hbm, idx_v, rows_v, sem):
        wid = lax.axis_index("s") * info.num_cores + lax.axis_index("c")
        base = wid * bpw

        def body(g, carry):
            start = base + g * ch
            pltpu.sync_copy(idx_hbm.at[pl.ds(start, ch)], idx_v)
            pltpu.async_copy(table_hbm.at[idx_v], rows_v, sem).wait()
            pltpu.sync_copy(rows_v, out_hbm.at[pl.ds(start, ch)])
            return carry

        lax.fori_loop(0, bpw // ch, body, 0)

    return k(table, idx)


def kernel(latents, prototypes):
    latents = latents.astype(jnp.float32)
    prototypes = prototypes.astype(jnp.float32)
    n = latents.shape[0]
    ln = jnp.sum(latents ** 2, axis=1, keepdims=True)
    pn = jnp.sum(prototypes ** 2, axis=1)[None, :]
    idx2d, loss2d = _tc_call(latents, ln, pn, prototypes)
    idx = idx2d.reshape(n)
    quantized = _sc_gather(prototypes, idx)
    return quantized, loss2d.reshape(())
